# baseline (speedup 1.0000x reference)
import functools

import jax
import jax.numpy as jnp
from jax.experimental import pallas as pl
from jax.experimental.pallas import tpu as pltpu
from jax.experimental.pallas import tpu_sc as plsc

_K = 32


def _head_kernel(q_ref, k_ref, p_ref, idx_ref):
    q = q_ref[0]
    k = k_ref[0]
    s = jax.lax.dot_general(q, k, (((1,), (1,)), ((), ())),
                            preferred_element_type=jnp.float32)
    sq, n_kv = s.shape
    lane = jax.lax.broadcasted_iota(jnp.int32, s.shape, 1)
    rank_lane = jax.lax.broadcasted_iota(jnp.int32, (sq, _K), 1)
    neg = jnp.float32(-jnp.inf)

    def body(t, carry):
        s, topv, topi = carry
        mu = jnp.max(s, axis=1, keepdims=True)
        eq = s == mu
        idx = jnp.min(jnp.where(eq, lane, n_kv), axis=1, keepdims=True)
        kill = eq & (lane == idx)
        s = jnp.where(kill, neg, s)
        topv = jnp.where(rank_lane == t, mu, topv)
        topi = jnp.where(rank_lane == t, idx, topi)
        return s, topv, topi

    topv0 = jnp.full((sq, _K), neg, jnp.float32)
    topi0 = jnp.zeros((sq, _K), jnp.int32)
    _, topv, topi = jax.lax.fori_loop(0, _K, body, (s, topv0, topi0))

    e = jnp.exp(topv - topv[:, 0:1])
    p = e / jnp.sum(e, axis=1, keepdims=True)
    p_ref[0] = p
    idx_ref[0] = topi


def _select_probs(Qf, Kf):
    G, Sq, D = Qf.shape
    Skv = Kf.shape[1]
    return pl.pallas_call(
        _head_kernel,
        grid=(G,),
        in_specs=[pl.BlockSpec((1, Sq, D), lambda g: (g, 0, 0)),
                  pl.BlockSpec((1, Skv, D), lambda g: (g, 0, 0))],
        out_specs=[pl.BlockSpec((1, Sq, _K), lambda g: (g, 0, 0)),
                   pl.BlockSpec((1, Sq, _K), lambda g: (g, 0, 0))],
        out_shape=[jax.ShapeDtypeStruct((G, Sq, _K), jnp.float32),
                   jax.ShapeDtypeStruct((G, Sq, _K), jnp.int32)],
    )(Qf, Kf)


def _sc_gather_matmul(G, Skv, D):
    hp = G // 32
    hd = _K * _K
    vstride = Skv * D

    mesh = plsc.VectorSubcoreMesh(core_axis_name="c", subcore_axis_name="s")

    @functools.partial(
        pl.kernel, mesh=mesh,
        out_type=jax.ShapeDtypeStruct((G * hd,), jnp.float32),
        scratch_types=[
            pltpu.VMEM((hd,), jnp.int32),
            pltpu.VMEM((8, 128), jnp.int32),
            pltpu.VMEM((8, 128), jnp.float32),
            pltpu.VMEM((hd,), jnp.float32),
            pltpu.VMEM((hd,), jnp.float32),
            pltpu.SemaphoreType.DMA,
        ],
    )
    def sc_kernel(p_hbm, idx_hbm, v_hbm, out_hbm,
                  idx_v, fidx_v, tv_v, p_v, out_v, sem):
        wid = jax.lax.axis_index("s") * 2 + jax.lax.axis_index("c")
        iota = jax.lax.iota(jnp.int32, 16)

        def head_body(h, carry):
            head = wid * hp + h
            base = head * hd
            pltpu.sync_copy(idx_hbm.at[pl.ds(base, hd)], idx_v)
            pltpu.sync_copy(p_hbm.at[pl.ds(base, hd)], p_v)
            voff = head * vstride
            for v in range(64):
                r, c0 = v // 8, (v % 8) * 16
                n0 = (v % 2) * 16
                sl = idx_v[pl.ds(v * 16, 16)]
                fidx_v[r, pl.ds(c0, 16)] = sl * D + (iota + (n0 + voff))
            cps = [pltpu.async_copy(v_hbm.at[fidx_v.at[r]], tv_v.at[r], sem)
                   for r in range(8)]
            for cp in cps:
                cp.wait()
            def row_body(i, carry):
                acc0 = jnp.zeros((16,), jnp.float32)
                acc1 = jnp.zeros((16,), jnp.float32)
                dnums = jax.lax.GatherDimensionNumbers(
                    offset_dims=(), collapsed_slice_dims=(0,),
                    start_index_map=(0,))
                for gq in range(2):
                    pvec = p_v[pl.ds(i * _K + gq * 16, 16)]
                    for jj in range(16):
                        j = gq * 16 + jj
                        pj = jax.lax.gather(
                            pvec, jnp.full((16, 1), jj, jnp.int32), dnums,
                            (1,), mode=jax.lax.GatherScatterMode.PROMISE_IN_BOUNDS)
                        t0 = tv_v[j // 4, pl.ds((j % 4) * 32, 16)]
                        t1 = tv_v[j // 4, pl.ds((j % 4) * 32 + 16, 16)]
                        acc0 = acc0 + pj * t0
                        acc1 = acc1 + pj * t1
                out_v[pl.ds(i * _K, 16)] = acc0
                out_v[pl.ds(i * _K + 16, 16)] = acc1
                return carry
            jax.lax.fori_loop(0, _K, row_body, 0)
            pltpu.sync_copy(out_v, out_hbm.at[pl.ds(base, hd)])
            return carry

        jax.lax.fori_loop(0, hp, head_body, 0)

    return sc_kernel


def kernel(Q, K, V):
    B, H, Sq, D = Q.shape
    Skv = K.shape[2]
    G = B * H
    Qf = Q.reshape(G, Sq, D)
    Kf = K.reshape(G, Skv, D)
    p, idx = _select_probs(Qf, Kf)
    sc = _sc_gather_matmul(G, Skv, D)
    out = sc(p.reshape(G * _K * _K), idx.reshape(G * _K * _K),
             V.reshape(G * Skv * D))
    return out.reshape(B, H, Sq, _K)
